# Initial kernel scaffold; baseline (speedup 1.0000x reference)
import jax
import jax.numpy as jnp
from jax.experimental import pallas as pl


def kernel(data, indices):
    raise NotImplementedError("write your pallas kernel here")


# speedup vs baseline: 1.1907x; 1.1907x over previous
import functools

import jax
import jax.numpy as jnp
from jax import lax
from jax.experimental import pallas as pl
from jax.experimental.pallas import tpu as pltpu
from jax.experimental.pallas import tpu_sc as plsc

NC, NS, LANES = 2, 16, 16
NW = NC * NS

B, N, L, D = 4096, 200, 50, 128
TOTAL = B * L
PER_W = TOTAL // NW
CHUNK = 128
NCHUNK = PER_W // CHUNK
BATCH_PER_W = PER_W // L

MAGIC = 83887


def _make_mesh():
    return plsc.VectorSubcoreMesh(
        core_axis_name="c", subcore_axis_name="s",
        num_cores=NC, num_subcores=NS)


@functools.partial(
    pl.kernel,
    out_type=jax.ShapeDtypeStruct((TOTAL, D), jnp.float32),
    mesh=_make_mesh(),
    scratch_types=[
        pltpu.VMEM((NCHUNK, CHUNK), jnp.int32),
        pltpu.VMEM((2, CHUNK, D), jnp.float32),
        pltpu.SemaphoreType.DMA,
        pltpu.SemaphoreType.DMA,
    ],
)
def _sc_gather(data_hbm, idx_hbm, out_hbm, idx_v, rows_v, gsem, osem):
    w = lax.axis_index("s") * NC + lax.axis_index("c")
    out_base = w * PER_W
    batch_base = w * BATCH_PER_W

    pltpu.sync_copy(idx_hbm.at[w], idx_v)

    def fix(t, _):
        q = t * LANES + lax.iota(jnp.int32, LANES)
        b = lax.shift_right_logical(q * MAGIC, 22)
        row = lax.shift_right_logical(t, 3)
        col = pl.multiple_of((t & 7) * LANES, LANES)
        idx_v[row, pl.ds(col, LANES)] = (
            idx_v[row, pl.ds(col, LANES)] + (batch_base + b) * N)
        return ()

    lax.fori_loop(0, PER_W // LANES, fix, (), unroll=8)

    def start_gather(j, buf):
        return pltpu.async_copy(data_hbm.at[idx_v.at[j]], rows_v.at[buf], gsem)

    start_gather(0, 0).wait()

    def step(j, _):
        buf = j & 1
        nxt = buf ^ 1
        gather = start_gather(j + 1, nxt)
        out_off = pl.multiple_of(out_base + j * CHUNK, CHUNK)
        put = pltpu.async_copy(
            rows_v.at[buf], out_hbm.at[pl.ds(out_off, CHUNK)], osem)
        gather.wait()
        put.wait()
        return ()

    lax.fori_loop(0, NCHUNK - 1, step, ())

    last = NCHUNK - 1
    pltpu.sync_copy(
        rows_v.at[last & 1],
        out_hbm.at[pl.ds(pl.multiple_of(out_base + last * CHUNK, CHUNK), CHUNK)])


def kernel(data, indices):
    data_flat = data.reshape(B * N, D)
    idx = indices.astype(jnp.int32).reshape(NW, NCHUNK, CHUNK)
    out_flat = _sc_gather(data_flat, idx)
    return out_flat.reshape(B, L, D)


# speedup vs baseline: 1.2916x; 1.0848x over previous
import functools

import jax
import jax.numpy as jnp
from jax import lax
from jax.experimental import pallas as pl
from jax.experimental.pallas import tpu as pltpu
from jax.experimental.pallas import tpu_sc as plsc

NC, NS, LANES = 2, 16, 16
NW = NC * NS

B, N, L, D = 4096, 200, 50, 128
TOTAL = B * L
PER_W = TOTAL // NW
CHUNK = 128
NCHUNK = PER_W // CHUNK
BATCH_PER_W = PER_W // L

MAGIC = 83887


def _make_mesh():
    return plsc.VectorSubcoreMesh(
        core_axis_name="c", subcore_axis_name="s",
        num_cores=NC, num_subcores=NS)


NBUF = 4


@functools.partial(
    pl.kernel,
    out_type=jax.ShapeDtypeStruct((TOTAL, D), jnp.float32),
    mesh=_make_mesh(),
    scratch_types=[
        pltpu.VMEM((NCHUNK, CHUNK), jnp.int32),
        pltpu.VMEM((NBUF, CHUNK, D), jnp.float32),
        pltpu.SemaphoreType.DMA((NBUF,)),
        pltpu.SemaphoreType.DMA((NBUF,)),
    ],
)
def _sc_gather(data_hbm, idx_hbm, out_hbm, idx_v, rows_v, gsem, osem):
    w = lax.axis_index("s") * NC + lax.axis_index("c")
    out_base = w * PER_W
    batch_base = w * BATCH_PER_W

    pltpu.sync_copy(idx_hbm.at[w], idx_v)

    def fix_chunk(j):
        for v in range(CHUNK // LANES):
            q = j * CHUNK + v * LANES + lax.iota(jnp.int32, LANES)
            b = lax.shift_right_logical(q * MAGIC, 22)
            sl = pl.ds(v * LANES, LANES)
            idx_v[j, sl] = idx_v[j, sl] + (batch_base + b) * N

    def start_gather(j, buf):
        pltpu.async_copy(data_hbm.at[idx_v.at[j]], rows_v.at[buf], gsem.at[buf])

    def wait_gather(buf):
        pltpu.make_async_copy(
            data_hbm.at[pl.ds(0, CHUNK)], rows_v.at[buf], gsem.at[buf]).wait()

    def wait_put(buf):
        pltpu.make_async_copy(
            rows_v.at[buf], out_hbm.at[pl.ds(0, CHUNK)], osem.at[buf]).wait()

    for b in range(NBUF):
        fix_chunk(b)
        start_gather(b, b)

    def step(j, _):
        buf = j & (NBUF - 1)
        wait_gather(buf)
        out_off = pl.multiple_of(out_base + j * CHUNK, CHUNK)
        pltpu.async_copy(
            rows_v.at[buf], out_hbm.at[pl.ds(out_off, CHUNK)], osem.at[buf])

        @pl.when(j + NBUF < NCHUNK)
        def _():
            wait_put(buf)
            fix_chunk(j + NBUF)
            start_gather(j + NBUF, buf)

        return ()

    lax.fori_loop(0, NCHUNK, step, ())

    for b in range(NBUF):
        wait_put((NCHUNK - NBUF + b) & (NBUF - 1))


def kernel(data, indices):
    data_flat = data.reshape(B * N, D)
    idx = indices.astype(jnp.int32).reshape(NW, NCHUNK, CHUNK)
    out_flat = _sc_gather(data_flat, idx)
    return out_flat.reshape(B, L, D)


# speedup vs baseline: 1.8831x; 1.4579x over previous
import functools

import jax
import jax.numpy as jnp
from jax import lax
from jax.experimental import pallas as pl
from jax.experimental.pallas import tpu as pltpu
from jax.experimental.pallas import tpu_sc as plsc

NC, NS, LANES = 2, 16, 16
NW = NC * NS

B, N, L, D = 4096, 200, 50, 128
LPAD = 64
BATCH_PER_W = B // NW
NCHUNK = BATCH_PER_W // 2
NBUF = 4


def _make_mesh():
    return plsc.VectorSubcoreMesh(
        core_axis_name="c", subcore_axis_name="s",
        num_cores=NC, num_subcores=NS)


@functools.partial(
    pl.kernel,
    out_type=jax.ShapeDtypeStruct((B, L, D), jnp.float32),
    mesh=_make_mesh(),
    compiler_params=pltpu.CompilerParams(use_tc_tiling_on_sc=True),
    scratch_types=[
        pltpu.VMEM((NCHUNK, 2 * LPAD), jnp.int32),
        pltpu.VMEM((NBUF, 2 * LPAD, D), jnp.float32),
        pltpu.SemaphoreType.DMA((NBUF,)),
        pltpu.SemaphoreType.DMA((NBUF,)),
    ],
)
def _sc_gather(data_hbm, idx_hbm, out_hbm, idx_v, rows_v, gsem, osem):
    w = lax.axis_index("s") * NC + lax.axis_index("c")
    batch_base = w * BATCH_PER_W

    pltpu.sync_copy(idx_hbm.at[pl.ds(w * NCHUNK, NCHUNK)], idx_v)

    def fix_chunk(g):
        for v in range(2 * LPAD // LANES):
            b = batch_base + 2 * g + (v * LANES) // LPAD
            sl = pl.ds(v * LANES, LANES)
            idx_v[g, sl] = idx_v[g, sl] + b * N

    def start_gather(g, buf):
        pltpu.async_copy(data_hbm.at[idx_v.at[g]], rows_v.at[buf], gsem.at[buf])

    def wait_gather(buf):
        pltpu.make_async_copy(
            data_hbm.at[pl.ds(0, 2 * LPAD)], rows_v.at[buf],
            gsem.at[buf]).wait()

    def start_puts(g, buf):
        b0 = batch_base + 2 * g
        pltpu.async_copy(
            rows_v.at[buf, pl.ds(0, L)], out_hbm.at[b0], osem.at[buf])
        pltpu.async_copy(
            rows_v.at[buf, pl.ds(LPAD, L)], out_hbm.at[b0 + 1], osem.at[buf])

    def wait_puts(buf):
        for _ in range(2):
            pltpu.make_async_copy(
                rows_v.at[buf, pl.ds(0, L)], out_hbm.at[0],
                osem.at[buf]).wait()

    for g in range(NBUF):
        fix_chunk(g)
        start_gather(g, g)

    def step(g, _):
        buf = g & (NBUF - 1)
        wait_gather(buf)
        start_puts(g, buf)

        @pl.when(g + NBUF < NCHUNK)
        def _():
            wait_puts(buf)
            fix_chunk(g + NBUF)
            start_gather(g + NBUF, buf)

        return ()

    lax.fori_loop(0, NCHUNK, step, ())

    for b in range(NBUF):
        wait_puts(b)


def kernel(data, indices):
    data_flat = data.reshape(B * N, D)
    idx = indices.astype(jnp.int32)
    idx_pad = jnp.concatenate(
        [idx, jnp.broadcast_to(idx[:, :1], (B, LPAD - L))], axis=1)
    idx_rows = idx_pad.reshape(B * LPAD // 128, 128)
    return _sc_gather(data_flat, idx_rows)


# speedup vs baseline: 4.0862x; 2.1700x over previous
import functools

import jax
import jax.numpy as jnp
from jax import lax
from jax.experimental import pallas as pl
from jax.experimental.pallas import tpu as pltpu
from jax.experimental.pallas import tpu_sc as plsc

NC, NS, LANES = 2, 16, 16
NW = NC * NS

B, N, L, D = 4096, 200, 50, 128
BATCH_PER_W = B // NW
PER_W = L * BATCH_PER_W
CHUNK = 128
NBUF = 4


def _make_mesh():
    return plsc.VectorSubcoreMesh(
        core_axis_name="c", subcore_axis_name="s",
        num_cores=NC, num_subcores=NS)


@functools.partial(
    pl.kernel,
    out_type=jax.ShapeDtypeStruct((L * B, D), jnp.float32),
    mesh=_make_mesh(),
    compiler_params=pltpu.CompilerParams(use_tc_tiling_on_sc=True),
    scratch_types=[
        pltpu.VMEM((L, CHUNK), jnp.int32),
        pltpu.VMEM((NBUF, CHUNK, D), jnp.float32),
        pltpu.SemaphoreType.DMA((NBUF,)),
        pltpu.SemaphoreType.DMA((NBUF,)),
    ],
)
def _sc_gather(data_hbm, idx_hbm, out_hbm, idx_v, rows_v, gsem, osem):
    w = lax.axis_index("s") * NC + lax.axis_index("c")
    batch_base = w * BATCH_PER_W

    pltpu.sync_copy(idx_hbm.at[w], idx_v)

    lane = lax.iota(jnp.int32, LANES)

    def fix_chunk(l):
        for v in range(CHUNK // LANES):
            b = batch_base + v * LANES + lane
            sl = pl.ds(v * LANES, LANES)
            idx_v[l, sl] = idx_v[l, sl] + b * N

    def start_gather(l, buf):
        pltpu.async_copy(
            data_hbm.at[idx_v.at[l]],
            rows_v.at[buf], gsem.at[buf])

    def wait_gather(buf):
        pltpu.make_async_copy(
            data_hbm.at[pl.ds(0, CHUNK)], rows_v.at[buf], gsem.at[buf]).wait()

    def start_put(l, buf):
        out_off = pl.multiple_of(l * B + batch_base, CHUNK)
        pltpu.async_copy(
            rows_v.at[buf], out_hbm.at[pl.ds(out_off, CHUNK)], osem.at[buf])

    def wait_put(buf):
        pltpu.make_async_copy(
            rows_v.at[buf], out_hbm.at[pl.ds(0, CHUNK)], osem.at[buf]).wait()

    for l in range(NBUF):
        fix_chunk(l)
        start_gather(l, l)

    def step(l, _):
        buf = l & (NBUF - 1)
        wait_gather(buf)
        start_put(l, buf)

        @pl.when(l + NBUF < L)
        def _():
            wait_put(buf)
            fix_chunk(l + NBUF)
            start_gather(l + NBUF, buf)

        return ()

    lax.fori_loop(0, L, step, ())

    for b in range(NBUF):
        wait_put(b)


def kernel(data, indices):
    data_flat = data.reshape(B * N, D)
    idx_blocks = (indices.astype(jnp.int32).T
                  .reshape(L, NW, BATCH_PER_W)
                  .transpose(1, 0, 2)
                  .reshape(NW, L, CHUNK))
    out_flat = _sc_gather(data_flat, idx_blocks)
    return out_flat.reshape(L, B, D).transpose(1, 0, 2)


# speedup vs baseline: 4.1362x; 1.0122x over previous
import functools

import jax
import jax.numpy as jnp
from jax import lax
from jax.experimental import pallas as pl
from jax.experimental.pallas import tpu as pltpu
from jax.experimental.pallas import tpu_sc as plsc

NC, NS, LANES = 2, 16, 16
NW = NC * NS

B, N, L, D = 4096, 200, 50, 128
BATCH_PER_W = B // NW
PER_W = L * BATCH_PER_W
CHUNK = 128
NBUF = 6


def _make_mesh():
    return plsc.VectorSubcoreMesh(
        core_axis_name="c", subcore_axis_name="s",
        num_cores=NC, num_subcores=NS)


@functools.partial(
    pl.kernel,
    out_type=jax.ShapeDtypeStruct((L * B, D), jnp.float32),
    mesh=_make_mesh(),
    compiler_params=pltpu.CompilerParams(use_tc_tiling_on_sc=True),
    scratch_types=[
        pltpu.VMEM((L, CHUNK), jnp.int32),
        pltpu.VMEM((NBUF, CHUNK, D), jnp.float32),
        pltpu.SemaphoreType.DMA((NBUF,)),
        pltpu.SemaphoreType.DMA((NBUF,)),
    ],
)
def _sc_gather(data_hbm, idx_hbm, out_hbm, idx_v, rows_v, gsem, osem):
    w = lax.axis_index("s") * NC + lax.axis_index("c")
    batch_base = w * BATCH_PER_W

    pltpu.sync_copy(idx_hbm.at[w], idx_v)

    lane = lax.iota(jnp.int32, LANES)

    def fix_chunk(l):
        for v in range(CHUNK // LANES):
            b = batch_base + v * LANES + lane
            sl = pl.ds(v * LANES, LANES)
            idx_v[l, sl] = idx_v[l, sl] + b * N

    def start_gather(l, buf):
        pltpu.async_copy(
            data_hbm.at[idx_v.at[l]],
            rows_v.at[buf], gsem.at[buf])

    def wait_gather(buf):
        pltpu.make_async_copy(
            data_hbm.at[pl.ds(0, CHUNK)], rows_v.at[buf], gsem.at[buf]).wait()

    def start_put(l, buf):
        out_off = pl.multiple_of(l * B + batch_base, CHUNK)
        pltpu.async_copy(
            rows_v.at[buf], out_hbm.at[pl.ds(out_off, CHUNK)], osem.at[buf])

    def wait_put(buf):
        pltpu.make_async_copy(
            rows_v.at[buf], out_hbm.at[pl.ds(0, CHUNK)], osem.at[buf]).wait()

    for l in range(NBUF):
        fix_chunk(l)
        start_gather(l, l)

    def step(l, buf):
        wait_gather(buf)
        start_put(l, buf)

        @pl.when(l + NBUF < L)
        def _():
            wait_put(buf)
            fix_chunk(l + NBUF)
            start_gather(l + NBUF, buf)

        nxt = buf + 1
        return lax.select(nxt == NBUF, 0, nxt)

    lax.fori_loop(0, L, step, 0)

    for b in range(NBUF):
        wait_put(b)


def kernel(data, indices):
    data_flat = data.reshape(B * N, D)
    idx_blocks = (indices.astype(jnp.int32).T
                  .reshape(L, NW, BATCH_PER_W)
                  .transpose(1, 0, 2)
                  .reshape(NW, L, CHUNK))
    out_flat = _sc_gather(data_flat, idx_blocks)
    return out_flat.reshape(L, B, D).transpose(1, 0, 2)
